# baseline (speedup 1.0000x reference)
import functools

import jax
import jax.numpy as jnp
from jax import lax
from jax.experimental import pallas as pl
from jax.experimental.pallas import tpu as pltpu
from jax.experimental.pallas import tpu_sc as plsc

SEQ = 256
C = 1024
CH = 32
JC = SEQ // CH

_info = plsc.get_sparse_core_info()
_NC, _NS = _info.num_cores, _info.num_subcores
_NW = _NC * _NS
_ROWS = SEQ // _NW

_mesh = plsc.VectorSubcoreMesh(core_axis_name="c", subcore_axis_name="s")


@functools.partial(
    pl.kernel,
    mesh=_mesh,
    out_type=jax.ShapeDtypeStruct((SEQ, SEQ, 2 * C), jnp.float32),
    scratch_types=[
        pltpu.VMEM((2, CH, C), jnp.float32),
        pltpu.SemaphoreType.DMA,
        pltpu.SemaphoreType.DMA,
    ],
)
def _sc_embed(emb_hbm, out_hbm, ebuf, ein_sem, eout_sem):
    wid = lax.axis_index("s") * _NC + lax.axis_index("c")
    base = wid * _ROWS

    ein = [None, None]
    eouts = [[], []]
    ein[0] = pltpu.async_copy(emb_hbm.at[pl.ds(0, CH), :], ebuf.at[0], ein_sem)
    for jc in range(JC):
        ep = jc & 1
        ein[ep].wait()
        if jc + 1 < JC:
            for h in eouts[1 - ep]:
                h.wait()
            eouts[1 - ep] = []
            ein[1 - ep] = pltpu.async_copy(
                emb_hbm.at[pl.ds((jc + 1) * CH, CH), :], ebuf.at[1 - ep],
                ein_sem)
        for ii in range(_ROWS):
            eouts[ep].append(pltpu.async_copy(
                ebuf.at[ep],
                out_hbm.at[base + ii, pl.ds(jc * CH, CH), pl.ds(C, C)],
                eout_sem))
    for hs in eouts:
        for h in hs:
            h.wait()


_ND = 32
_DR = SEQ // _ND


def _tc_body(x_hbm, shell_hbm, out_hbm, sem):
    del shell_hbm
    copies = [
        pltpu.make_async_copy(
            x_hbm.at[pl.ds(k * _DR, _DR)],
            out_hbm.at[pl.ds(k * _DR, _DR), :, pl.ds(0, C)],
            sem,
        )
        for k in range(_ND)
    ]
    for cp in copies:
        cp.start()
    for cp in copies:
        cp.wait()


_tc_fill = pl.pallas_call(
    _tc_body,
    in_specs=[
        pl.BlockSpec(memory_space=pl.ANY),
        pl.BlockSpec(memory_space=pl.ANY),
    ],
    out_specs=pl.BlockSpec(memory_space=pl.ANY),
    out_shape=jax.ShapeDtypeStruct((SEQ, SEQ, 2 * C), jnp.float32),
    scratch_shapes=[pltpu.SemaphoreType.DMA],
    input_output_aliases={1: 0},
)


def kernel(x, embedding):
    shell = _sc_embed(embedding)
    return _tc_fill(x, shell)


# speedup vs baseline: 26.7329x; 26.7329x over previous
import functools

import jax
import jax.numpy as jnp
from jax import lax
from jax.experimental import pallas as pl
from jax.experimental.pallas import tpu as pltpu
from jax.experimental.pallas import tpu_sc as plsc

SEQ = 256
C = 1024
CH = 32
JC = SEQ // CH

_info = plsc.get_sparse_core_info()
_NC, _NS = _info.num_cores, _info.num_subcores
_NW = _NC * _NS
_ROWS = SEQ // _NW

_mesh = plsc.VectorSubcoreMesh(core_axis_name="c", subcore_axis_name="s")


@functools.partial(
    pl.kernel,
    mesh=_mesh,
    out_type=jax.ShapeDtypeStruct((SEQ, SEQ, 2 * C), jnp.float32),
    scratch_types=[
        pltpu.VMEM((3, CH, C), jnp.float32),
        pltpu.SemaphoreType.DMA,
        pltpu.SemaphoreType.DMA,
    ],
)
def _sc_embed(emb_hbm, out_hbm, ebuf, ein_sem, eout_sem):
    wid = lax.axis_index("s") * _NC + lax.axis_index("c")
    base = wid * _ROWS

    ein = [None, None, None]
    eouts = [[], [], []]
    for j in range(2):
        ein[j] = pltpu.async_copy(
            emb_hbm.at[pl.ds(j * CH, CH), :], ebuf.at[j], ein_sem)
    for jc in range(JC):
        b = jc % 3
        ein[b].wait()
        for ii in range(_ROWS):
            eouts[b].append(pltpu.async_copy(
                ebuf.at[b],
                out_hbm.at[base + ii, pl.ds(jc * CH, CH), pl.ds(C, C)],
                eout_sem))
        nj = jc + 2
        if nj < JC:
            nb = nj % 3
            for h in eouts[nb]:
                h.wait()
            eouts[nb] = []
            ein[nb] = pltpu.async_copy(
                emb_hbm.at[pl.ds(nj * CH, CH), :], ebuf.at[nb], ein_sem)
    for hs in eouts:
        for h in hs:
            h.wait()


def _tc_body(x_ref, shell_ref, out_ref):
    del shell_ref
    out_ref[...] = x_ref[...]


_BI = 8

_tc_fill = pl.pallas_call(
    _tc_body,
    grid=(SEQ // _BI,),
    in_specs=[
        pl.BlockSpec((_BI, SEQ, C), lambda i: (i, 0, 0)),
        pl.BlockSpec(memory_space=pl.ANY),
    ],
    out_specs=pl.BlockSpec((_BI, SEQ, C), lambda i: (i, 0, 0)),
    out_shape=jax.ShapeDtypeStruct((SEQ, SEQ, 2 * C), jnp.float32),
    input_output_aliases={1: 0},
)


def kernel(x, embedding):
    shell = _sc_embed(embedding)
    return _tc_fill(x, shell)
